# baseline (speedup 1.0000x reference)
import functools

import jax
import jax.numpy as jnp
from jax.experimental import pallas as pl
from jax.experimental.pallas import tpu as pltpu

NUM_LAYERS = 3
SEQ_LEN = 512
TOPK = 2
D_MODEL = 32
D_FF = 64
PATCHES = (16, 8, 32, 64)
NUM_EXPERTS = 4
FLAT = SEQ_LEN * D_MODEL


def _softplus(x):
    return jnp.maximum(x, 0.0) + jnp.log1p(jnp.exp(-jnp.abs(x)))


def _ndtr(x):
    return 0.5 * (1.0 + jax.lax.erf(x * 0.7071067811865476))


def _gating(flat, wgn, noise):
    logits = jnp.dot(flat, wgn, preferred_element_type=jnp.float32)
    return _gating_from_logits(logits, noise)


def _gating_from_logits(logits, noise):
    bb = logits.shape[0]
    clean = logits[:, :NUM_EXPERTS]
    noise_std = _softplus(logits[:, NUM_EXPERTS:]) + 1e-2
    noisy = clean + noise * noise_std

    idxs = jax.lax.broadcasted_iota(jnp.int32, (bb, NUM_EXPERTS), 1)
    big = jnp.int32(NUM_EXPERTS)

    def pick(v):
        m = jnp.max(v, axis=1, keepdims=True)
        eq = v == m
        first_idx = jnp.min(jnp.where(eq, idxs, big), axis=1, keepdims=True)
        return m, idxs == first_idx

    m0, oh0 = pick(noisy)
    v1 = jnp.where(oh0, -jnp.inf, noisy)
    m1, oh1 = pick(v1)
    v2 = jnp.where(oh1, -jnp.inf, v1)
    m2 = jnp.max(v2, axis=1, keepdims=True)

    e1 = jnp.exp(m1 - m0)
    denom = 1.0 + e1
    gates = jnp.where(oh0, 1.0 / denom, 0.0) + jnp.where(oh1, e1 / denom, 0.0)

    is_in = noisy > m2
    prob_in = _ndtr((clean - m2) / noise_std)
    prob_out = _ndtr((clean - m1) / noise_std)
    load_vec = jnp.sum(jnp.where(is_in, prob_in, prob_out), axis=0,
                       keepdims=True)
    imp_vec = jnp.sum(gates, axis=0, keepdims=True)
    return gates, imp_vec, load_vec


def _prep_kernel(nb, x_ref, w_ref, b_ref, wgc_ref, goff_ref, noise_ref,
                 out_ref, mean_ref, std_ref, gates_ref, imp_ref, load_ref,
                 imp_s, load_s):
    i = pl.program_id(0)

    @pl.when(i == 0)
    def _init():
        imp_s[...] = jnp.zeros_like(imp_s)
        load_s[...] = jnp.zeros_like(load_s)

    x = x_ref[...]
    m = jnp.mean(x, axis=1, keepdims=True)
    v = jnp.mean((x - m) ** 2, axis=1, keepdims=True)
    s = jnp.sqrt(v + 1e-5)
    xn = (x - m) / s
    mean_ref[...] = m
    std_ref[...] = s
    out_ref[...] = xn[:, :, None] * w_ref[...][None, :, :] + b_ref[...][None, :, :]

    gates, imp_vec, load_vec = _gating_from_logits(
        jnp.dot(xn, wgc_ref[...], preferred_element_type=jnp.float32)
        + goff_ref[...], noise_ref[...])
    gates_ref[...] = gates
    imp_s[...] += imp_vec
    load_s[...] += load_vec

    @pl.when(i == nb - 1)
    def _fin():
        imp_ref[...] = imp_s[...]
        load_ref[...] = load_s[...]


def _expert_kernel(v0, v1, v2, v3, *wrefs_and_outs):
    wrefs = wrefs_and_outs[:3 * NUM_EXPERTS]
    outs = wrefs_and_outs[3 * NUM_EXPERTS:]
    views = (v0, v1, v2, v3)
    for e, p in enumerate(PATCHES):
        w1 = wrefs[3 * e][...]
        b1 = wrefs[3 * e + 1][...]
        w2 = wrefs[3 * e + 2][...]
        xv = views[e][...]
        h = jnp.dot(xv, w1, preferred_element_type=jnp.float32)
        h = jax.nn.gelu(h + b1)
        outs[e][...] = jnp.dot(h, w2, preferred_element_type=jnp.float32)


def _combine_kernel(nb, has_next, s_ref, y0, y1, y2, y3, g_ref, b2_ref,
                    wgn_ref, noise_ref,
                    out_ref, gates_ref, imp_ref, load_ref, imp_s, load_s):
    i = pl.program_id(0)

    @pl.when(i == 0)
    def _init():
        imp_s[...] = jnp.zeros_like(imp_s)
        load_s[...] = jnp.zeros_like(load_s)

    g = g_ref[...]
    bias = jnp.dot(g, b2_ref[...], preferred_element_type=jnp.float32)
    state = (s_ref[...] + bias
             + g[:, 0:1] * y0[...] + g[:, 1:2] * y1[...]
             + g[:, 2:3] * y2[...] + g[:, 3:4] * y3[...])
    out_ref[...] = state

    if has_next:
        gates, imp_vec, load_vec = _gating(state, wgn_ref[...], noise_ref[...])
        gates_ref[...] = gates
        imp_s[...] += imp_vec
        load_s[...] += load_vec

        @pl.when(i == nb - 1)
        def _fin():
            imp_ref[...] = imp_s[...]
            load_ref[...] = load_s[...]


def _proj_kernel(flat_ref, pw_ref, pb_ref, mean_ref, std_ref, out_ref):
    y = jnp.dot(flat_ref[...], pw_ref[...], preferred_element_type=jnp.float32)
    y = y + pb_ref[...]
    out_ref[...] = y * std_ref[...] + mean_ref[...]


def _loss_kernel(imp_ref, load_ref, out_ref):
    eps = 1e-10
    imp = imp_ref[...]
    load = load_ref[...]
    im = jnp.mean(imp, axis=1, keepdims=True)
    iv = jnp.mean((imp - im) ** 2, axis=1, keepdims=True)
    lm = jnp.mean(load, axis=1, keepdims=True)
    lv = jnp.mean((load - lm) ** 2, axis=1, keepdims=True)
    cv = iv / (im * im + eps) + lv / (lm * lm + eps)
    out_ref[...] = jnp.sum(cv).reshape(1, 1)


def _cspec(shape):
    return pl.BlockSpec(shape, lambda i: tuple(0 for _ in shape))


def kernel(x, params):
    B = x.shape[0]
    x2 = x[..., 0]
    f32 = jnp.float32

    wgn = [jnp.concatenate([params['l%d_gate' % l], params['l%d_noise' % l]],
                           axis=1) for l in range(NUM_LAYERS)]
    nkey = jax.random.key(1234)
    noise = [jax.random.normal(jax.random.fold_in(nkey, l), (B, NUM_EXPERTS),
                               f32) for l in range(NUM_LAYERS)]
    b2s = [jnp.stack([jnp.tile(params['l%d_e%d_b2' % (l, e)], SEQ_LEN // p)
                      for e, p in enumerate(PATCHES)])
           for l in range(NUM_LAYERS)]

    wgc = jnp.einsum('d,lde->le', params['start_W'][0],
                     wgn[0].reshape(SEQ_LEN, D_MODEL, 2 * NUM_EXPERTS))
    goff = jnp.einsum('d,lde->e', params['start_b'],
                      wgn[0].reshape(SEQ_LEN, D_MODEL, 2 * NUM_EXPERTS))
    BA = 32
    na = B // BA
    out3, mean, std, gates0, imp0, load0 = pl.pallas_call(
        functools.partial(_prep_kernel, na),
        grid=(na,),
        in_specs=[
            pl.BlockSpec((BA, SEQ_LEN), lambda i: (i, 0)),
            _cspec((1, D_MODEL)),
            _cspec((1, D_MODEL)),
            _cspec((SEQ_LEN, 2 * NUM_EXPERTS)),
            _cspec((1, 2 * NUM_EXPERTS)),
            pl.BlockSpec((BA, NUM_EXPERTS), lambda i: (i, 0)),
        ],
        out_specs=[
            pl.BlockSpec((BA, SEQ_LEN, D_MODEL), lambda i: (i, 0, 0)),
            pl.BlockSpec((BA, 1), lambda i: (i, 0)),
            pl.BlockSpec((BA, 1), lambda i: (i, 0)),
            pl.BlockSpec((BA, NUM_EXPERTS), lambda i: (i, 0)),
            _cspec((1, NUM_EXPERTS)),
            _cspec((1, NUM_EXPERTS)),
        ],
        out_shape=[
            jax.ShapeDtypeStruct((B, SEQ_LEN, D_MODEL), f32),
            jax.ShapeDtypeStruct((B, 1), f32),
            jax.ShapeDtypeStruct((B, 1), f32),
            jax.ShapeDtypeStruct((B, NUM_EXPERTS), f32),
            jax.ShapeDtypeStruct((1, NUM_EXPERTS), f32),
            jax.ShapeDtypeStruct((1, NUM_EXPERTS), f32),
        ],
        scratch_shapes=[pltpu.VMEM((1, NUM_EXPERTS), f32)] * 2,
        compiler_params=pltpu.CompilerParams(
            dimension_semantics=("arbitrary",)),
    )(x2, params['start_W'], params['start_b'].reshape(1, D_MODEL),
      wgc, goff.reshape(1, 2 * NUM_EXPERTS), noise[0])

    state = out3.reshape(B, FLAT)
    gates = gates0
    imps, loads = [imp0], [load0]

    BB = 32
    nb = B // BB
    for l in range(NUM_LAYERS):
        e_in_specs = []
        e_ins = []
        for p in PATCHES:
            n = SEQ_LEN // p
            e_in_specs.append(
                pl.BlockSpec((BB * n, p * D_MODEL), lambda i: (i, 0)))
            e_ins.append(state.reshape(B * n, p * D_MODEL))
        y_specs, y_shapes = [], []
        for e, p in enumerate(PATCHES):
            n = SEQ_LEN // p
            e_in_specs += [_cspec((p * D_MODEL, D_FF)), _cspec((1, D_FF)),
                           _cspec((D_FF, p * D_MODEL))]
            e_ins += [params['l%d_e%d_W1' % (l, e)],
                      params['l%d_e%d_b1' % (l, e)].reshape(1, D_FF),
                      params['l%d_e%d_W2' % (l, e)]]
            y_specs.append(
                pl.BlockSpec((BB * n, p * D_MODEL), lambda i: (i, 0)))
            y_shapes.append(jax.ShapeDtypeStruct((B * n, p * D_MODEL), f32))
        ys = pl.pallas_call(
            _expert_kernel,
            grid=(nb,),
            in_specs=e_in_specs,
            out_specs=y_specs,
            out_shape=y_shapes,
            compiler_params=pltpu.CompilerParams(
                dimension_semantics=("arbitrary",)),
        )(*e_ins)

        has_next = l + 1 < NUM_LAYERS
        ln = l + 1 if has_next else 0
        outs = pl.pallas_call(
            functools.partial(_combine_kernel, nb, has_next),
            grid=(nb,),
            in_specs=[pl.BlockSpec((BB, FLAT), lambda i: (i, 0))]
            + [pl.BlockSpec((BB, FLAT), lambda i: (i, 0))] * NUM_EXPERTS
            + [pl.BlockSpec((BB, NUM_EXPERTS), lambda i: (i, 0)),
               _cspec((NUM_EXPERTS, FLAT)),
               _cspec((FLAT, 2 * NUM_EXPERTS)),
               pl.BlockSpec((BB, NUM_EXPERTS), lambda i: (i, 0))],
            out_specs=[
                pl.BlockSpec((BB, FLAT), lambda i: (i, 0)),
                pl.BlockSpec((BB, NUM_EXPERTS), lambda i: (i, 0)),
                _cspec((1, NUM_EXPERTS)),
                _cspec((1, NUM_EXPERTS)),
            ],
            out_shape=[
                jax.ShapeDtypeStruct((B, FLAT), f32),
                jax.ShapeDtypeStruct((B, NUM_EXPERTS), f32),
                jax.ShapeDtypeStruct((1, NUM_EXPERTS), f32),
                jax.ShapeDtypeStruct((1, NUM_EXPERTS), f32),
            ],
            scratch_shapes=[pltpu.VMEM((1, NUM_EXPERTS), f32)] * 2,
            compiler_params=pltpu.CompilerParams(
                dimension_semantics=("arbitrary",)),
        )(state, *[ys[e].reshape(B, FLAT) for e in range(NUM_EXPERTS)],
          gates, b2s[l], wgn[ln], noise[ln])
        state, gates = outs[0], outs[1]
        if has_next:
            imps.append(outs[2])
            loads.append(outs[3])

    BP = 32
    ni = B // BP
    proj = pl.pallas_call(
        _proj_kernel,
        grid=(ni,),
        in_specs=[
            pl.BlockSpec((BP, FLAT), lambda i: (i, 0)),
            _cspec((FLAT, SEQ_LEN)),
            _cspec((1, SEQ_LEN)),
            pl.BlockSpec((BP, 1), lambda i: (i, 0)),
            pl.BlockSpec((BP, 1), lambda i: (i, 0)),
        ],
        out_specs=pl.BlockSpec((BP, SEQ_LEN), lambda i: (i, 0)),
        out_shape=jax.ShapeDtypeStruct((B, SEQ_LEN), f32),
        compiler_params=pltpu.CompilerParams(
            dimension_semantics=("arbitrary",)),
    )(state, params['proj_W'], params['proj_b'].reshape(1, SEQ_LEN),
      mean, std)

    loss = pl.pallas_call(
        _loss_kernel,
        out_shape=jax.ShapeDtypeStruct((1, 1), f32),
    )(jnp.concatenate(imps, axis=0), jnp.concatenate(loads, axis=0))

    return proj.reshape(B, SEQ_LEN, 1), state, loss.reshape(())


# speedup vs baseline: 3.2104x; 3.2104x over previous
import functools

import jax
import jax.numpy as jnp
from jax.experimental import pallas as pl
from jax.experimental.pallas import tpu as pltpu

NUM_LAYERS = 3
SEQ_LEN = 512
TOPK = 2
D_MODEL = 32
D_FF = 64
PATCHES = (16, 8, 32, 64)
NUM_EXPERTS = 4
FLAT = SEQ_LEN * D_MODEL
Q = 8
W64 = FLAT // Q


def _softplus(x):
    return jnp.maximum(x, 0.0) + jnp.log1p(jnp.exp(-jnp.abs(x)))


def _ndtr(x):
    return 0.5 * (1.0 + jax.lax.erf(x * 0.7071067811865476))


def _gating_from_logits(logits, noise):
    bb = logits.shape[0]
    clean = logits[:, :NUM_EXPERTS]
    noise_std = _softplus(logits[:, NUM_EXPERTS:]) + 1e-2
    noisy = clean + noise * noise_std

    idxs = jax.lax.broadcasted_iota(jnp.int32, (bb, NUM_EXPERTS), 1)
    big = jnp.int32(NUM_EXPERTS)

    def pick(v):
        m = jnp.max(v, axis=1, keepdims=True)
        eq = v == m
        first_idx = jnp.min(jnp.where(eq, idxs, big), axis=1, keepdims=True)
        return m, idxs == first_idx

    m0, oh0 = pick(noisy)
    v1 = jnp.where(oh0, -jnp.inf, noisy)
    m1, oh1 = pick(v1)
    v2 = jnp.where(oh1, -jnp.inf, v1)
    m2 = jnp.max(v2, axis=1, keepdims=True)

    e1 = jnp.exp(m1 - m0)
    denom = 1.0 + e1
    gates = jnp.where(oh0, 1.0 / denom, 0.0) + jnp.where(oh1, e1 / denom, 0.0)

    is_in = noisy > m2
    prob_in = _ndtr((clean - m2) / noise_std)
    prob_out = _ndtr((clean - m1) / noise_std)
    load_vec = jnp.sum(jnp.where(is_in, prob_in, prob_out), axis=0,
                       keepdims=True)
    imp_vec = jnp.sum(gates, axis=0, keepdims=True)
    return gates, imp_vec, load_vec


def _prep_kernel(x_ref, w_ref, b_ref, out_ref, mean_ref, std_ref):
    x = x_ref[...]
    m = jnp.mean(x, axis=1, keepdims=True)
    v = jnp.mean((x - m) ** 2, axis=1, keepdims=True)
    s = jnp.sqrt(v + 1e-5)
    xn = (x - m) / s
    mean_ref[...] = m
    std_ref[...] = s
    out_ref[...] = xn[:, :, None] * w_ref[...][None, :, :] + b_ref[...][None, :, :]


def _backbone_kernel(nb, bb, *refs):
    state_in = refs[0]
    wgn_refs = refs[1:1 + NUM_LAYERS]
    noise_refs = refs[1 + NUM_LAYERS:1 + 2 * NUM_LAYERS]
    b2_refs = refs[1 + 2 * NUM_LAYERS:1 + 3 * NUM_LAYERS]
    ew = refs[1 + 3 * NUM_LAYERS:1 + 3 * NUM_LAYERS
              + 3 * NUM_EXPERTS * NUM_LAYERS]
    out_ref, loss_ref, imp_s, load_s = refs[-4:]

    i = pl.program_id(0)

    @pl.when(i == 0)
    def _init():
        imp_s[...] = jnp.zeros_like(imp_s)
        load_s[...] = jnp.zeros_like(load_s)

    f32 = jnp.float32
    rows = bb * Q
    state = state_in[...]

    gmask = (jax.lax.broadcasted_iota(jnp.int32, (Q, Q * 8), 1) // 8
             == jax.lax.broadcasted_iota(jnp.int32, (Q, Q * 8), 0)
             ).astype(f32)
    fold = (jax.lax.broadcasted_iota(jnp.int32, (Q * 8, 8), 0) % 8
            == jax.lax.broadcasted_iota(jnp.int32, (Q * 8, 8), 1)
            ).astype(f32)
    samp = (jax.lax.broadcasted_iota(jnp.int32, (rows, bb), 0) // Q
            == jax.lax.broadcasted_iota(jnp.int32, (rows, bb), 1)
            ).astype(f32)
    rexp = (jax.lax.broadcasted_iota(jnp.int32, (NUM_EXPERTS, 8 * NUM_EXPERTS),
                                     1) // 8
            == jax.lax.broadcasted_iota(jnp.int32,
                                        (NUM_EXPERTS, 8 * NUM_EXPERTS), 0)
            ).astype(f32)
    m8 = (jax.lax.broadcasted_iota(jnp.int32, (rows, 8 * NUM_EXPERTS), 1) % 8
          == jax.lax.broadcasted_iota(jnp.int32, (rows, 8 * NUM_EXPERTS), 0)
          % Q).astype(f32)

    for l in range(NUM_LAYERS):
        g_all = jnp.dot(state, wgn_refs[l][...],
                        preferred_element_type=f32)
        g3 = g_all.reshape(bb, Q, Q * 8) * gmask[None]
        logits = jnp.dot(jnp.sum(g3, axis=1), fold,
                         preferred_element_type=f32)
        gates, imp_vec, load_vec = _gating_from_logits(
            logits, noise_refs[l][...])
        imp_s[l:l + 1, :] += imp_vec
        load_s[l:l + 1, :] += load_vec

        g_col = jnp.dot(samp, gates, preferred_element_type=f32)
        gx = jnp.dot(g_col, rexp, preferred_element_type=f32) * m8
        acc = state + jnp.dot(gx, b2_refs[l][...],
                              preferred_element_type=f32)

        for e, p in enumerate(PATCHES):
            w1, b1, w2 = ew[(l * NUM_EXPERTS + e) * 3:
                            (l * NUM_EXPERTS + e) * 3 + 3]
            kd = p * D_MODEL
            n = SEQ_LEN // p
            segs = n // Q
            xv = state.reshape(bb * n, kd)
            h = jnp.dot(xv, w1[...], preferred_element_type=f32)
            h = jax.nn.gelu(h + b1[...])
            ge = g_col[:, e:e + 1]
            if segs == 1:
                acc = acc + jnp.dot(h * ge, w2[...],
                                    preferred_element_type=f32)
            else:
                h3 = h.reshape(rows, segs, D_FF)
                y64 = jnp.concatenate(
                    [jnp.dot(h3[:, s, :] * ge, w2[...],
                             preferred_element_type=f32)
                     for s in range(segs)], axis=1)
                acc = acc + y64
        state = acc

    out_ref[...] = state

    @pl.when(i == nb - 1)
    def _fin():
        eps = 1e-10
        imp = imp_s[...]
        load = load_s[...]
        im = jnp.mean(imp, axis=1, keepdims=True)
        iv = jnp.mean((imp - im) ** 2, axis=1, keepdims=True)
        lm = jnp.mean(load, axis=1, keepdims=True)
        lv = jnp.mean((load - lm) ** 2, axis=1, keepdims=True)
        cv = iv / (im * im + eps) + lv / (lm * lm + eps)
        loss_ref[...] = jnp.sum(cv).reshape(1, 1)


def _proj_kernel(flat_ref, pw_ref, pb_ref, mean_ref, std_ref, out_ref):
    y = jnp.dot(flat_ref[...], pw_ref[...], preferred_element_type=jnp.float32)
    y = y + pb_ref[...]
    out_ref[...] = y * std_ref[...] + mean_ref[...]


def _cspec(shape):
    return pl.BlockSpec(shape, lambda i: tuple(0 for _ in shape))


def kernel(x, params):
    B = x.shape[0]
    x2 = x[..., 0]
    f32 = jnp.float32

    BA = 32
    na = B // BA
    out3, mean, std = pl.pallas_call(
        _prep_kernel,
        grid=(na,),
        in_specs=[
            pl.BlockSpec((BA, SEQ_LEN), lambda i: (i, 0)),
            _cspec((1, D_MODEL)),
            _cspec((1, D_MODEL)),
        ],
        out_specs=[
            pl.BlockSpec((BA, SEQ_LEN, D_MODEL), lambda i: (i, 0, 0)),
            pl.BlockSpec((BA, 1), lambda i: (i, 0)),
            pl.BlockSpec((BA, 1), lambda i: (i, 0)),
        ],
        out_shape=[
            jax.ShapeDtypeStruct((B, SEQ_LEN, D_MODEL), f32),
            jax.ShapeDtypeStruct((B, 1), f32),
            jax.ShapeDtypeStruct((B, 1), f32),
        ],
    )(x2, params['start_W'], params['start_b'].reshape(1, D_MODEL))
    state0 = out3.reshape(B * Q, W64)

    wgn_bc, noise, b2cat = [], [], []
    nkey = jax.random.key(1234)
    for l in range(NUM_LAYERS):
        wgn = jnp.concatenate([params['l%d_gate' % l],
                               params['l%d_noise' % l]], axis=1)
        wgn_bc.append(wgn.reshape(Q, W64, 2 * NUM_EXPERTS)
                      .transpose(1, 0, 2).reshape(W64, Q * 8))
        noise.append(jax.random.normal(jax.random.fold_in(nkey, l),
                                       (B, NUM_EXPERTS), f32))
        b2cat.append(jnp.stack(
            [jnp.tile(params['l%d_e%d_b2' % (l, e)], SEQ_LEN // p)
             for e, p in enumerate(PATCHES)]).reshape(NUM_EXPERTS * Q, W64))

    expert_ws = []
    for l in range(NUM_LAYERS):
        for e, p in enumerate(PATCHES):
            expert_ws += [
                params['l%d_e%d_W1' % (l, e)],
                params['l%d_e%d_b1' % (l, e)].reshape(1, D_FF),
                params['l%d_e%d_W2' % (l, e)],
            ]

    BB = 32
    nb = B // BB
    in_specs = [pl.BlockSpec((BB * Q, W64), lambda i: (i, 0))]
    in_specs += [_cspec((W64, Q * 8))] * NUM_LAYERS
    in_specs += [pl.BlockSpec((BB, NUM_EXPERTS), lambda i: (i, 0))] * NUM_LAYERS
    in_specs += [_cspec((NUM_EXPERTS * Q, W64))] * NUM_LAYERS
    for l in range(NUM_LAYERS):
        for e, p in enumerate(PATCHES):
            in_specs += [
                _cspec((p * D_MODEL, D_FF)),
                _cspec((1, D_FF)),
                _cspec((D_FF, p * D_MODEL)),
            ]

    state3, loss = pl.pallas_call(
        functools.partial(_backbone_kernel, nb, BB),
        grid=(nb,),
        in_specs=in_specs,
        out_specs=[
            pl.BlockSpec((BB * Q, W64), lambda i: (i, 0)),
            pl.BlockSpec((1, 1), lambda i: (0, 0)),
        ],
        out_shape=[
            jax.ShapeDtypeStruct((B * Q, W64), f32),
            jax.ShapeDtypeStruct((1, 1), f32),
        ],
        scratch_shapes=[
            pltpu.VMEM((NUM_LAYERS, NUM_EXPERTS), f32),
            pltpu.VMEM((NUM_LAYERS, NUM_EXPERTS), f32),
        ],
        compiler_params=pltpu.CompilerParams(
            dimension_semantics=("arbitrary",)),
    )(state0, *wgn_bc, *noise, *b2cat, *expert_ws)
    flat3 = state3.reshape(B, FLAT)

    BP = 32
    ni = B // BP
    proj = pl.pallas_call(
        _proj_kernel,
        grid=(ni,),
        in_specs=[
            pl.BlockSpec((BP, FLAT), lambda i: (i, 0)),
            _cspec((FLAT, SEQ_LEN)),
            _cspec((1, SEQ_LEN)),
            pl.BlockSpec((BP, 1), lambda i: (i, 0)),
            pl.BlockSpec((BP, 1), lambda i: (i, 0)),
        ],
        out_specs=pl.BlockSpec((BP, SEQ_LEN), lambda i: (i, 0)),
        out_shape=jax.ShapeDtypeStruct((B, SEQ_LEN), f32),
        compiler_params=pltpu.CompilerParams(
            dimension_semantics=("arbitrary",)),
    )(flat3, params['proj_W'], params['proj_b'].reshape(1, SEQ_LEN),
      mean, std)

    return proj.reshape(B, SEQ_LEN, 1), flat3, loss.reshape(())


# speedup vs baseline: 4.4513x; 1.3865x over previous
import functools

import jax
import jax.numpy as jnp
from jax.experimental import pallas as pl
from jax.experimental.pallas import tpu as pltpu

NUM_LAYERS = 3
SEQ_LEN = 512
TOPK = 2
D_MODEL = 32
D_FF = 64
PATCHES = (16, 8, 32, 64)
NUM_EXPERTS = 4
FLAT = SEQ_LEN * D_MODEL
Q = 8
W64 = FLAT // Q


def _softplus(x):
    return jnp.maximum(x, 0.0) + jnp.log1p(jnp.exp(-jnp.abs(x)))


def _ndtr(x):
    return 0.5 * (1.0 + jax.lax.erf(x * 0.7071067811865476))


def _gating_from_logits(logits, noise):
    bb = logits.shape[0]
    clean = logits[:, :NUM_EXPERTS]
    noise_std = _softplus(logits[:, NUM_EXPERTS:]) + 1e-2
    noisy = clean + noise * noise_std

    idxs = jax.lax.broadcasted_iota(jnp.int32, (bb, NUM_EXPERTS), 1)
    big = jnp.int32(NUM_EXPERTS)

    def pick(v):
        m = jnp.max(v, axis=1, keepdims=True)
        eq = v == m
        first_idx = jnp.min(jnp.where(eq, idxs, big), axis=1, keepdims=True)
        return m, idxs == first_idx

    m0, oh0 = pick(noisy)
    v1 = jnp.where(oh0, -jnp.inf, noisy)
    m1, oh1 = pick(v1)
    v2 = jnp.where(oh1, -jnp.inf, v1)
    m2 = jnp.max(v2, axis=1, keepdims=True)

    e1 = jnp.exp(m1 - m0)
    denom = 1.0 + e1
    gates = jnp.where(oh0, 1.0 / denom, 0.0) + jnp.where(oh1, e1 / denom, 0.0)

    is_in = noisy > m2
    prob_in = _ndtr((clean - m2) / noise_std)
    prob_out = _ndtr((clean - m1) / noise_std)
    load_vec = jnp.sum(jnp.where(is_in, prob_in, prob_out), axis=0,
                       keepdims=True)
    imp_vec = jnp.sum(gates, axis=0, keepdims=True)
    return gates, imp_vec, load_vec


def _prep_kernel(ba, x_ref, k_ref, b_ref, out_ref, mean_ref, std_ref):
    f32 = jnp.float32
    rows = ba * Q
    lq = SEQ_LEN // Q
    x8 = x_ref[...]
    sampT = (jax.lax.broadcasted_iota(jnp.int32, (ba, rows), 1) // Q
             == jax.lax.broadcasted_iota(jnp.int32, (ba, rows), 0)
             ).astype(f32)
    samp = (jax.lax.broadcasted_iota(jnp.int32, (rows, ba), 0) // Q
            == jax.lax.broadcasted_iota(jnp.int32, (rows, ba), 1)
            ).astype(f32)
    ones = jnp.ones((lq, 1), f32)
    m = jnp.dot(sampT, jnp.dot(x8, ones, preferred_element_type=f32),
                preferred_element_type=f32) / SEQ_LEN
    xc = x8 - jnp.dot(samp, m, preferred_element_type=f32)
    v = jnp.dot(sampT, jnp.dot(xc * xc, ones, preferred_element_type=f32),
                preferred_element_type=f32) / SEQ_LEN
    s = jnp.sqrt(v + 1e-5)
    xn = xc * jnp.dot(samp, 1.0 / s, preferred_element_type=f32)
    mean_ref[...] = m
    std_ref[...] = s
    out_ref[...] = (jnp.dot(xn, k_ref[...], preferred_element_type=f32)
                    + b_ref[...])


def _backbone_kernel(nb, bb, *refs):
    state_in = refs[0]
    wgn_refs = refs[1:1 + NUM_LAYERS]
    noise_refs = refs[1 + NUM_LAYERS:1 + 2 * NUM_LAYERS]
    b2_refs = refs[1 + 2 * NUM_LAYERS:1 + 3 * NUM_LAYERS]
    ew = refs[1 + 3 * NUM_LAYERS:1 + 3 * NUM_LAYERS
              + 3 * NUM_EXPERTS * NUM_LAYERS]
    out_ref, loss_ref, imp_s, load_s = refs[-4:]

    i = pl.program_id(0)

    @pl.when(i == 0)
    def _init():
        imp_s[...] = jnp.zeros_like(imp_s)
        load_s[...] = jnp.zeros_like(load_s)

    f32 = jnp.float32
    rows = bb * Q
    state = state_in[...]

    gmask = (jax.lax.broadcasted_iota(jnp.int32, (Q, Q * 8), 1) // 8
             == jax.lax.broadcasted_iota(jnp.int32, (Q, Q * 8), 0)
             ).astype(f32)
    fold = (jax.lax.broadcasted_iota(jnp.int32, (Q * 8, 8), 0) % 8
            == jax.lax.broadcasted_iota(jnp.int32, (Q * 8, 8), 1)
            ).astype(f32)
    samp = (jax.lax.broadcasted_iota(jnp.int32, (rows, bb), 0) // Q
            == jax.lax.broadcasted_iota(jnp.int32, (rows, bb), 1)
            ).astype(f32)

    for l in range(NUM_LAYERS):
        g_all = jnp.dot(state, wgn_refs[l][...],
                        preferred_element_type=f32)
        g3 = g_all.reshape(bb, Q, Q * 8) * gmask[None]
        logits = jnp.dot(jnp.sum(g3, axis=1), fold,
                         preferred_element_type=f32)
        gates, imp_vec, load_vec = _gating_from_logits(
            logits, noise_refs[l][...])
        imp_s[l:l + 1, :] += imp_vec
        load_s[l:l + 1, :] += load_vec

        g_col = jnp.dot(samp, gates, preferred_element_type=f32)
        acc = state + jnp.dot(g_col, b2_refs[l][...],
                              preferred_element_type=f32)

        for e, p in enumerate(PATCHES):
            w1, b1, w2 = ew[(l * NUM_EXPERTS + e) * 3:
                            (l * NUM_EXPERTS + e) * 3 + 3]
            pw = p * D_MODEL
            nseg = W64 // pw
            ge = g_col[:, e:e + 1]
            parts = []
            for s in range(nseg):
                xs = state[:, s * pw:(s + 1) * pw] if nseg > 1 else state
                h = jnp.dot(xs, w1[...], preferred_element_type=f32)
                h = jax.nn.gelu(h + b1[...])
                parts.append(jnp.dot(h * ge, w2[...],
                                     preferred_element_type=f32))
            y64 = parts[0] if nseg == 1 else jnp.concatenate(parts, axis=1)
            acc = acc + y64
        state = acc

    out_ref[...] = state

    @pl.when(i == nb - 1)
    def _fin():
        eps = 1e-10
        imp = imp_s[...]
        load = load_s[...]
        im = jnp.mean(imp, axis=1, keepdims=True)
        iv = jnp.mean((imp - im) ** 2, axis=1, keepdims=True)
        lm = jnp.mean(load, axis=1, keepdims=True)
        lv = jnp.mean((load - lm) ** 2, axis=1, keepdims=True)
        cv = iv / (im * im + eps) + lv / (lm * lm + eps)
        loss_ref[...] = jnp.sum(cv).reshape(1, 1)


def _proj_kernel(flat_ref, pw_ref, pb_ref, mean_ref, std_ref, out_ref):
    y = jnp.dot(flat_ref[...], pw_ref[...], preferred_element_type=jnp.float32)
    y = y + pb_ref[...]
    out_ref[...] = y * std_ref[...] + mean_ref[...]


def _cspec(shape):
    return pl.BlockSpec(shape, lambda i: tuple(0 for _ in shape))


def kernel(x, params):
    B = x.shape[0]
    x2 = x[..., 0]
    f32 = jnp.float32

    BA = 32
    na = B // BA
    lq = SEQ_LEN // Q
    x8 = x2.reshape(B * Q, lq)
    kmat = jnp.kron(jnp.eye(lq, dtype=f32),
                    params['start_W'].reshape(1, D_MODEL))
    brow = jnp.tile(params['start_b'].reshape(1, D_MODEL), (1, lq))
    state0, mean, std = pl.pallas_call(
        functools.partial(_prep_kernel, BA),
        grid=(na,),
        in_specs=[
            pl.BlockSpec((BA * Q, lq), lambda i: (i, 0)),
            _cspec((lq, W64)),
            _cspec((1, W64)),
        ],
        out_specs=[
            pl.BlockSpec((BA * Q, W64), lambda i: (i, 0)),
            pl.BlockSpec((BA, 1), lambda i: (i, 0)),
            pl.BlockSpec((BA, 1), lambda i: (i, 0)),
        ],
        out_shape=[
            jax.ShapeDtypeStruct((B * Q, W64), f32),
            jax.ShapeDtypeStruct((B, 1), f32),
            jax.ShapeDtypeStruct((B, 1), f32),
        ],
    )(x8, kmat, brow)

    wgn_bc, noise, b2cat = [], [], []
    nkey = jax.random.key(1234)
    for l in range(NUM_LAYERS):
        wgn = jnp.concatenate([params['l%d_gate' % l],
                               params['l%d_noise' % l]], axis=1)
        wgn_bc.append(wgn.reshape(Q, W64, 2 * NUM_EXPERTS)
                      .transpose(1, 0, 2).reshape(W64, Q * 8))
        noise.append(jax.random.normal(jax.random.fold_in(nkey, l),
                                       (B, NUM_EXPERTS), f32))
        b2cat.append(jnp.stack(
            [jnp.tile(params['l%d_e%d_b2' % (l, e)], W64 // (p * D_MODEL))
             for e, p in enumerate(PATCHES)]).reshape(NUM_EXPERTS, W64))

    expert_ws = []
    for l in range(NUM_LAYERS):
        for e, p in enumerate(PATCHES):
            expert_ws += [
                params['l%d_e%d_W1' % (l, e)],
                params['l%d_e%d_b1' % (l, e)].reshape(1, D_FF),
                params['l%d_e%d_W2' % (l, e)],
            ]

    BB = 32
    nb = B // BB
    in_specs = [pl.BlockSpec((BB * Q, W64), lambda i: (i, 0))]
    in_specs += [_cspec((W64, Q * 8))] * NUM_LAYERS
    in_specs += [pl.BlockSpec((BB, NUM_EXPERTS), lambda i: (i, 0))] * NUM_LAYERS
    in_specs += [_cspec((NUM_EXPERTS, W64))] * NUM_LAYERS
    for l in range(NUM_LAYERS):
        for e, p in enumerate(PATCHES):
            in_specs += [
                _cspec((p * D_MODEL, D_FF)),
                _cspec((1, D_FF)),
                _cspec((D_FF, p * D_MODEL)),
            ]

    state3, loss = pl.pallas_call(
        functools.partial(_backbone_kernel, nb, BB),
        grid=(nb,),
        in_specs=in_specs,
        out_specs=[
            pl.BlockSpec((BB * Q, W64), lambda i: (i, 0)),
            pl.BlockSpec((1, 1), lambda i: (0, 0)),
        ],
        out_shape=[
            jax.ShapeDtypeStruct((B * Q, W64), f32),
            jax.ShapeDtypeStruct((1, 1), f32),
        ],
        scratch_shapes=[
            pltpu.VMEM((NUM_LAYERS, NUM_EXPERTS), f32),
            pltpu.VMEM((NUM_LAYERS, NUM_EXPERTS), f32),
        ],
        compiler_params=pltpu.CompilerParams(
            dimension_semantics=("arbitrary",)),
    )(state0, *wgn_bc, *noise, *b2cat, *expert_ws)
    flat3 = state3.reshape(B, FLAT)

    BP = 32
    ni = B // BP
    proj = pl.pallas_call(
        _proj_kernel,
        grid=(ni,),
        in_specs=[
            pl.BlockSpec((BP, FLAT), lambda i: (i, 0)),
            _cspec((FLAT, SEQ_LEN)),
            _cspec((1, SEQ_LEN)),
            pl.BlockSpec((BP, 1), lambda i: (i, 0)),
            pl.BlockSpec((BP, 1), lambda i: (i, 0)),
        ],
        out_specs=pl.BlockSpec((BP, SEQ_LEN), lambda i: (i, 0)),
        out_shape=jax.ShapeDtypeStruct((B, SEQ_LEN), f32),
        compiler_params=pltpu.CompilerParams(
            dimension_semantics=("arbitrary",)),
    )(flat3, params['proj_W'], params['proj_b'].reshape(1, SEQ_LEN),
      mean, std)

    return proj.reshape(B, SEQ_LEN, 1), flat3, loss.reshape(())
